# baseline (speedup 1.0000x reference)
import jax
import jax.numpy as jnp
from jax import lax
from jax.experimental import pallas as pl
from jax.experimental.pallas import tpu as pltpu
from jax.experimental.pallas import tpu_sc as plsc

B, C, N, K, OUT = 8, 64, 1024, 20, 128
RT = 2
RTS = N // RT
NK = N * K
M = B * N * K
NC, NS = 2, 16
NW = NC * NS
OPW = OUT // NW
CHUNK = 2560
NCHUNK = NK // CHUNK
NEG = -3.0e38


def _tc1_body(x_ref, xt_ref, w_ref, idxt_ref, u_ref, v_ref, d_ref, ic_ref,
              g_ref, bits_ref):
  rt = pl.program_id(1)
  x = x_ref[0]
  xr = xt_ref[0]
  xrt = jnp.transpose(xr, (1, 0))
  xtx = lax.dot_general(xrt, x, (((1,), (0,)), ((), ())),
                        preferred_element_type=jnp.float32)
  sq = jnp.sum(x * x, axis=0, keepdims=True)
  d_ref[...] = 2.0 * xtx - sq
  NCH = N // 128
  g = d_ref[:, 0:128]
  for c in range(1, NCH):
    g = jnp.maximum(g, d_ref[:, c * 128:(c + 1) * 128])
  g_ref[...] = g
  bits_ref[...] = jnp.zeros((RTS, 128), jnp.int32)
  lane128 = lax.broadcasted_iota(jnp.int32, (RTS, 128), 1)
  chunk8 = lax.broadcasted_iota(jnp.int32, (RTS, NCH), 1)
  for k in range(K):
    l2 = jnp.argmax(g_ref[...], axis=1).astype(jnp.int32).reshape(RTS, 1)
    cj = jnp.concatenate(
        [jnp.take_along_axis(d_ref[:, c * 128:(c + 1) * 128], l2, axis=1,
                             mode="promise_in_bounds")
         for c in range(NCH)], axis=1)
    bt = jnp.take_along_axis(bits_ref[...], l2, axis=1,
                             mode="promise_in_bounds")
    cjm = jnp.where(((bt >> chunk8) & 1) == 1, NEG, cj)
    j2 = jnp.argmax(cjm, axis=1).astype(jnp.int32).reshape(RTS, 1)
    ic_ref[:, k:k + 1] = j2 * 128 + l2
    nb = bt | (1 << j2)
    bits_ref[...] = jnp.where(lane128 == l2, nb, bits_ref[...])
    m2 = jnp.max(jnp.where(chunk8 == j2, NEG, cjm), axis=1, keepdims=True)
    g_ref[...] = jnp.where(lane128 == l2, m2, g_ref[...])
  ict = jnp.transpose(ic_ref[...], (1, 0))
  idxt_ref[0] = ict[:24]

  @pl.when(rt == 0)
  def _():
    w1 = w_ref[:, :C]
    w2 = w_ref[:, C:]
    u_ref[0] = lax.dot_general(w1, x, (((1,), (0,)), ((), ())),
                               preferred_element_type=jnp.float32)
    v_ref[0] = lax.dot_general(w2 - w1, x, (((1,), (0,)), ((), ())),
                               preferred_element_type=jnp.float32)


def _tc1(x, W):
  return pl.pallas_call(
      _tc1_body,
      grid=(B, RT),
      in_specs=[
          pl.BlockSpec((1, C, N), lambda b, rt: (b, 0, 0)),
          pl.BlockSpec((1, C, RTS), lambda b, rt: (b, 0, rt)),
          pl.BlockSpec((OUT, 2 * C), lambda b, rt: (0, 0)),
      ],
      out_specs=[
          pl.BlockSpec((1, 24, RTS), lambda b, rt: (b, 0, rt)),
          pl.BlockSpec((1, OUT, N), lambda b, rt: (b, 0, 0)),
          pl.BlockSpec((1, OUT, N), lambda b, rt: (b, 0, 0)),
      ],
      out_shape=[
          jax.ShapeDtypeStruct((B, 24, N), jnp.int32),
          jax.ShapeDtypeStruct((B, OUT, N), jnp.float32),
          jax.ShapeDtypeStruct((B, OUT, N), jnp.float32),
      ],
      scratch_shapes=[
          pltpu.VMEM((RTS, N), jnp.float32),
          pltpu.VMEM((RTS, 128), jnp.int32),
          pltpu.VMEM((RTS, 128), jnp.float32),
          pltpu.VMEM((RTS, 128), jnp.int32),
      ],
  )(x, x, W)


def _sc_mesh():
  return plsc.VectorSubcoreMesh(core_axis_name="c", subcore_axis_name="s",
                                num_cores=NC, num_subcores=NS)


def _sca_body(u_hbm, v_hbm, ixt_hbm, p_hbm, u_v, v_v, ix_v, st_v):
  w = lax.axis_index("s") * NC + lax.axis_index("c")
  acc = tuple(jnp.zeros((16,), jnp.float32) for _ in range(5 * OPW))
  for b in range(B):
    pltpu.sync_copy(u_hbm.at[b, pl.ds(w * (OPW * N), OPW * N)], u_v)
    pltpu.sync_copy(v_hbm.at[b, pl.ds(w * (OPW * N), OPW * N)], v_v)
    pltpu.sync_copy(ixt_hbm.at[b, pl.ds(0, K * N)], ix_v)

    def n16_body(i, a):
      s = [jnp.zeros((16,), jnp.float32) for _ in range(OPW)]
      q = [jnp.zeros((16,), jnp.float32) for _ in range(OPW)]
      for k in range(K):
        iv = ix_v[pl.ds(k * N + i * 16, 16)]
        for o in range(OPW):
          g = plsc.load_gather(u_v, [iv + jnp.int32(o * N)])
          s[o] = s[o] + g
          q[o] = q[o] + g * g
      a = list(a)
      for o in range(OPW):
        vv = v_v[pl.ds(o * N + i * 16, 16)]
        a[5 * o + 0] = a[5 * o + 0] + s[o]
        a[5 * o + 1] = a[5 * o + 1] + q[o]
        a[5 * o + 2] = a[5 * o + 2] + vv * s[o]
        a[5 * o + 3] = a[5 * o + 3] + vv
        a[5 * o + 4] = a[5 * o + 4] + vv * vv
      return tuple(a)

    acc = plsc.parallel_loop(0, N // 16, carry=acc, unroll=4)(
        lambda i, a: n16_body(i, a))
  for j in range(5 * OPW):
    st_v[pl.ds(j * 16, 16)] = acc[j]
  pltpu.sync_copy(st_v, p_hbm.at[w])


def _sc_a(u, v, ixt):
  f = pl.kernel(
      _sca_body,
      out_type=jax.ShapeDtypeStruct((NW, 5 * OPW * 16), jnp.float32),
      mesh=_sc_mesh(),
      compiler_params=pltpu.CompilerParams(needs_layout_passes=False),
      scratch_types=[
          pltpu.VMEM((OPW * N,), jnp.float32),
          pltpu.VMEM((OPW * N,), jnp.float32),
          pltpu.VMEM((K * N,), jnp.int32),
          pltpu.VMEM((5 * OPW * 16,), jnp.float32),
      ],
  )
  return f(u, v, ixt)


def _rsqrt_nr(x):
  i = lax.bitcast_convert_type(x, jnp.int32)
  y = lax.bitcast_convert_type(jnp.int32(0x5F3759DF) - (i >> 1), jnp.float32)
  for _ in range(3):
    y = y * (1.5 - 0.5 * x * y * y)
  return y


OPB = 8
KH = K // 2


def _scb_body(u_hbm, v_hbm, ixt_hbm, p_hbm, gb_hbm,
              y_hbm, u_v, v_v, ix_v, p_v, gb_v, st_v, sem0, sem1):
  w = lax.axis_index("s") * NC + lax.axis_index("c")
  o8 = w % 16
  kh = w // 16
  pltpu.sync_copy(p_hbm.at[pl.ds(o8 * 640, 640)], p_v)
  pltpu.sync_copy(gb_hbm, gb_v)
  ssum = [jnp.sum(p_v[pl.ds(j * 16, 16)]) for j in range(5 * OPB)]
  scale = []
  shift = []
  rm = 1.0 / float(M)
  for o in range(OPB):
    s_s, s_q, s_vs, s_v, s_v2 = (ssum[5 * o + j] for j in range(5))
    mean = (s_s + K * s_v) * rm
    ey2 = (s_q + 2.0 * s_vs + K * s_v2) * rm
    var = ey2 - mean * mean
    ch = o8 * OPB + jnp.full((16,), o, jnp.int32)
    g_sp = plsc.load_gather(gb_v, [ch])
    b_sp = plsc.load_gather(gb_v, [ch + jnp.int32(OUT)])
    rs = _rsqrt_nr(jnp.full((16,), 1e-5, jnp.float32) + var)
    sc = g_sp * rs
    scale.append(sc)
    shift.append(b_sp - sc * mean)

  sems = (sem0, sem1)

  def b_body(b, car):
    pltpu.sync_copy(u_hbm.at[b, pl.ds(o8 * (OPB * N), OPB * N)], u_v)
    pltpu.sync_copy(v_hbm.at[b, pl.ds(o8 * (OPB * N), OPB * N)], v_v)
    pltpu.sync_copy(ixt_hbm.at[b, pl.ds(kh * (KH * N), KH * N)], ix_v)

    for o in range(OPB):
      @plsc.parallel_loop(0, N // 16, unroll=2)
      def pre_body(i, o=o):
        sl = pl.ds(o * N + i * 16, 16)
        u_v[sl] = u_v[sl] * scale[o]
        v_v[sl] = v_v[sl] * scale[o] + shift[o]

    for k in range(KH):
      buf = k % 2

      def drain(buf=buf):
        pltpu.make_async_copy(
            y_hbm.at[0, 0, pl.ds(o8 * OPB, OPB), :],
            st_v.at[pl.ds(buf * OPB, OPB)], sems[buf]).wait()

      if k >= 2:
        drain()
      else:
        @pl.when(b > 0)
        def _(drain=drain):
          drain()

      @plsc.parallel_loop(0, N // 16, unroll=4)
      def k_body(i, k=k, buf=buf):
        iv = ix_v[pl.ds(k * N + i * 16, 16)]
        for o in range(OPB):
          g = plsc.load_gather(u_v, [iv + jnp.int32(o * N)])
          y = g + v_v[pl.ds(o * N + i * 16, 16)]
          y = jnp.where(y > 0, y, 0.2 * y)
          st_v[buf * OPB + o, pl.ds(i * 16, 16)] = y
      pltpu.async_copy(
          st_v.at[pl.ds(buf * OPB, OPB)],
          y_hbm.at[b, kh * KH + k, pl.ds(o8 * OPB, OPB), :], sems[buf])
    return car

  lax.fori_loop(0, B, b_body, 0)
  for buf in range(2):
    pltpu.make_async_copy(
        y_hbm.at[0, 0, pl.ds(o8 * OPB, OPB), :],
        st_v.at[pl.ds(buf * OPB, OPB)], sems[buf]).wait()


def _sc_b(u, v, ixt, p, gb):
  f = pl.kernel(
      _scb_body,
      out_type=jax.ShapeDtypeStruct((B, K, OUT, N), jnp.float32),
      mesh=_sc_mesh(),
      compiler_params=pltpu.CompilerParams(needs_layout_passes=False),
      scratch_types=[
          pltpu.VMEM((OPB * N,), jnp.float32),
          pltpu.VMEM((OPB * N,), jnp.float32),
          pltpu.VMEM((KH * N,), jnp.int32),
          pltpu.VMEM((5 * OPB * 16,), jnp.float32),
          pltpu.VMEM((2 * OUT,), jnp.float32),
          pltpu.VMEM((2 * OPB, N), jnp.float32),
          pltpu.SemaphoreType.DMA,
          pltpu.SemaphoreType.DMA,
      ],
  )
  return f(u, v, ixt, p, gb)


def kernel(x, W, gamma, beta):
  idxt, u, v = _tc1(x, W)
  uf = u.reshape(B, OUT * N)
  vf = v.reshape(B, OUT * N)
  ixf = idxt.reshape(B, 24 * N)
  p = _sc_a(uf, vf, ixf)
  gb = jnp.concatenate([gamma, beta])
  y = _sc_b(uf, vf, ixf, p.reshape(NW * 5 * OPW * 16), gb)
  return jnp.transpose(y, (0, 2, 3, 1))


# speedup vs baseline: 2.3201x; 2.3201x over previous
import jax
import jax.numpy as jnp
from jax import lax
from jax.experimental import pallas as pl
from jax.experimental.pallas import tpu as pltpu
from jax.experimental.pallas import tpu_sc as plsc

B, C, N, K, OUT = 8, 64, 1024, 20, 128
RT = 8
RTS = N // RT
NK = N * K
M = B * N * K
NC, NS = 2, 16
NW = NC * NS
OPW = OUT // NW
CHUNK = 2560
NCHUNK = NK // CHUNK
NEG = -3.0e38


def _tc1_body(x_ref, xt_ref, w_ref, idxt_ref, u_ref, v_ref, d_ref, ic_ref):
  rt = pl.program_id(1)
  x = x_ref[0]
  xr = xt_ref[0]
  xrt = jnp.transpose(xr, (1, 0))
  xtx = lax.dot_general(xrt, x, (((1,), (0,)), ((), ())),
                        preferred_element_type=jnp.float32)
  sq = jnp.sum(x * x, axis=0, keepdims=True)
  d_ref[...] = 2.0 * xtx - sq
  lane = lax.broadcasted_iota(jnp.int32, (RTS, N), 1)
  for k in range(K):
    am = jnp.argmax(d_ref[...], axis=1).astype(jnp.int32)
    am2 = jnp.reshape(am, (RTS, 1))
    ic_ref[:, k:k + 1] = am2
    d_ref[...] = jnp.where(lane == am2, NEG, d_ref[...])
  ict = jnp.transpose(ic_ref[...], (1, 0))
  idxt_ref[0] = ict[:24]

  @pl.when(rt == 0)
  def _():
    w1 = w_ref[:, :C]
    w2 = w_ref[:, C:]
    u_ref[0] = lax.dot_general(w1, x, (((1,), (0,)), ((), ())),
                               preferred_element_type=jnp.float32)
    v_ref[0] = lax.dot_general(w2 - w1, x, (((1,), (0,)), ((), ())),
                               preferred_element_type=jnp.float32)


def _tc1(x, W):
  return pl.pallas_call(
      _tc1_body,
      grid=(B, RT),
      in_specs=[
          pl.BlockSpec((1, C, N), lambda b, rt: (b, 0, 0)),
          pl.BlockSpec((1, C, RTS), lambda b, rt: (b, 0, rt)),
          pl.BlockSpec((OUT, 2 * C), lambda b, rt: (0, 0)),
      ],
      out_specs=[
          pl.BlockSpec((1, 24, RTS), lambda b, rt: (b, 0, rt)),
          pl.BlockSpec((1, OUT, N), lambda b, rt: (b, 0, 0)),
          pl.BlockSpec((1, OUT, N), lambda b, rt: (b, 0, 0)),
      ],
      out_shape=[
          jax.ShapeDtypeStruct((B, 24, N), jnp.int32),
          jax.ShapeDtypeStruct((B, OUT, N), jnp.float32),
          jax.ShapeDtypeStruct((B, OUT, N), jnp.float32),
      ],
      scratch_shapes=[
          pltpu.VMEM((RTS, N), jnp.float32),
          pltpu.VMEM((RTS, 128), jnp.int32),
      ],
  )(x, x, W)


def _sc_mesh():
  return plsc.VectorSubcoreMesh(core_axis_name="c", subcore_axis_name="s",
                                num_cores=NC, num_subcores=NS)


def _sca_body(u_hbm, v_hbm, ixt_hbm, p_hbm, u_v, v_v, ix_v, st_v):
  w = lax.axis_index("s") * NC + lax.axis_index("c")
  acc = tuple(jnp.zeros((16,), jnp.float32) for _ in range(5 * OPW))
  for b in range(B):
    pltpu.sync_copy(u_hbm.at[b, pl.ds(w * (OPW * N), OPW * N)], u_v)
    pltpu.sync_copy(v_hbm.at[b, pl.ds(w * (OPW * N), OPW * N)], v_v)
    pltpu.sync_copy(ixt_hbm.at[b, pl.ds(0, K * N)], ix_v)

    def n16_body(i, a):
      s = [jnp.zeros((16,), jnp.float32) for _ in range(OPW)]
      q = [jnp.zeros((16,), jnp.float32) for _ in range(OPW)]
      for k in range(K):
        iv = ix_v[pl.ds(k * N + i * 16, 16)]
        for o in range(OPW):
          g = plsc.load_gather(u_v, [iv + jnp.int32(o * N)])
          s[o] = s[o] + g
          q[o] = q[o] + g * g
      a = list(a)
      for o in range(OPW):
        vv = v_v[pl.ds(o * N + i * 16, 16)]
        a[5 * o + 0] = a[5 * o + 0] + s[o]
        a[5 * o + 1] = a[5 * o + 1] + q[o]
        a[5 * o + 2] = a[5 * o + 2] + vv * s[o]
        a[5 * o + 3] = a[5 * o + 3] + vv
        a[5 * o + 4] = a[5 * o + 4] + vv * vv
      return tuple(a)

    acc = plsc.parallel_loop(0, N // 16, carry=acc, unroll=4)(
        lambda i, a: n16_body(i, a))
  for j in range(5 * OPW):
    st_v[pl.ds(j * 16, 16)] = acc[j]
  pltpu.sync_copy(st_v, p_hbm.at[w])


def _sc_a(u, v, ixt):
  f = pl.kernel(
      _sca_body,
      out_type=jax.ShapeDtypeStruct((NW, 5 * OPW * 16), jnp.float32),
      mesh=_sc_mesh(),
      compiler_params=pltpu.CompilerParams(needs_layout_passes=False),
      scratch_types=[
          pltpu.VMEM((OPW * N,), jnp.float32),
          pltpu.VMEM((OPW * N,), jnp.float32),
          pltpu.VMEM((K * N,), jnp.int32),
          pltpu.VMEM((5 * OPW * 16,), jnp.float32),
      ],
  )
  return f(u, v, ixt)


def _rsqrt_nr(x):
  i = lax.bitcast_convert_type(x, jnp.int32)
  y = lax.bitcast_convert_type(jnp.int32(0x5F3759DF) - (i >> 1), jnp.float32)
  for _ in range(3):
    y = y * (1.5 - 0.5 * x * y * y)
  return y


OPB = 8
KH = K // 2


def _scb_body(u_hbm, v_hbm, ixt_hbm, p_hbm, gb_hbm,
              y_hbm, u_v, v_v, ix_v, p_v, gb_v, st_v, sem0, sem1):
  w = lax.axis_index("s") * NC + lax.axis_index("c")
  o8 = w % 16
  kh = w // 16
  pltpu.sync_copy(p_hbm.at[pl.ds(o8 * 640, 640)], p_v)
  pltpu.sync_copy(gb_hbm, gb_v)
  ssum = [jnp.sum(p_v[pl.ds(j * 16, 16)]) for j in range(5 * OPB)]
  scale = []
  shift = []
  rm = 1.0 / float(M)
  for o in range(OPB):
    s_s, s_q, s_vs, s_v, s_v2 = (ssum[5 * o + j] for j in range(5))
    mean = (s_s + K * s_v) * rm
    ey2 = (s_q + 2.0 * s_vs + K * s_v2) * rm
    var = ey2 - mean * mean
    ch = o8 * OPB + jnp.full((16,), o, jnp.int32)
    g_sp = plsc.load_gather(gb_v, [ch])
    b_sp = plsc.load_gather(gb_v, [ch + jnp.int32(OUT)])
    rs = _rsqrt_nr(jnp.full((16,), 1e-5, jnp.float32) + var)
    sc = g_sp * rs
    scale.append(sc)
    shift.append(b_sp - sc * mean)

  sems = (sem0, sem1)

  def b_body(b, car):
    pltpu.sync_copy(u_hbm.at[b, pl.ds(o8 * (OPB * N), OPB * N)], u_v)
    pltpu.sync_copy(v_hbm.at[b, pl.ds(o8 * (OPB * N), OPB * N)], v_v)
    pltpu.sync_copy(ixt_hbm.at[b, pl.ds(kh * (KH * N), KH * N)], ix_v)

    for o in range(OPB):
      @plsc.parallel_loop(0, N // 16, unroll=2)
      def pre_body(i, o=o):
        sl = pl.ds(o * N + i * 16, 16)
        u_v[sl] = u_v[sl] * scale[o]
        v_v[sl] = v_v[sl] * scale[o] + shift[o]

    for k in range(KH):
      buf = k % 2

      def drain(buf=buf):
        pltpu.make_async_copy(
            y_hbm.at[0, 0, pl.ds(o8 * OPB, OPB), :],
            st_v.at[pl.ds(buf * OPB, OPB)], sems[buf]).wait()

      if k >= 2:
        drain()
      else:
        @pl.when(b > 0)
        def _(drain=drain):
          drain()

      @plsc.parallel_loop(0, N // 16, unroll=4)
      def k_body(i, k=k, buf=buf):
        iv = ix_v[pl.ds(k * N + i * 16, 16)]
        for o in range(OPB):
          g = plsc.load_gather(u_v, [iv + jnp.int32(o * N)])
          y = g + v_v[pl.ds(o * N + i * 16, 16)]
          y = jnp.where(y > 0, y, 0.2 * y)
          st_v[buf * OPB + o, pl.ds(i * 16, 16)] = y
      pltpu.async_copy(
          st_v.at[pl.ds(buf * OPB, OPB)],
          y_hbm.at[b, kh * KH + k, pl.ds(o8 * OPB, OPB), :], sems[buf])
    return car

  lax.fori_loop(0, B, b_body, 0)
  for buf in range(2):
    pltpu.make_async_copy(
        y_hbm.at[0, 0, pl.ds(o8 * OPB, OPB), :],
        st_v.at[pl.ds(buf * OPB, OPB)], sems[buf]).wait()


def _sc_b(u, v, ixt, p, gb):
  f = pl.kernel(
      _scb_body,
      out_type=jax.ShapeDtypeStruct((B, K, OUT, N), jnp.float32),
      mesh=_sc_mesh(),
      compiler_params=pltpu.CompilerParams(needs_layout_passes=False),
      scratch_types=[
          pltpu.VMEM((OPB * N,), jnp.float32),
          pltpu.VMEM((OPB * N,), jnp.float32),
          pltpu.VMEM((KH * N,), jnp.int32),
          pltpu.VMEM((5 * OPB * 16,), jnp.float32),
          pltpu.VMEM((2 * OUT,), jnp.float32),
          pltpu.VMEM((2 * OPB, N), jnp.float32),
          pltpu.SemaphoreType.DMA,
          pltpu.SemaphoreType.DMA,
      ],
  )
  return f(u, v, ixt, p, gb)


def kernel(x, W, gamma, beta):
  idxt, u, v = _tc1(x, W)
  uf = u.reshape(B, OUT * N)
  vf = v.reshape(B, OUT * N)
  ixf = idxt.reshape(B, 24 * N)
  p = _sc_a(uf, vf, ixf)
  gb = jnp.concatenate([gamma, beta])
  y = _sc_b(uf, vf, ixf, p.reshape(NW * 5 * OPW * 16), gb)
  return jnp.transpose(y, (0, 2, 3, 1))
